# baseline (speedup 1.0000x reference)
import functools

import jax
import jax.numpy as jnp
from jax import lax
from jax.experimental import pallas as pl
from jax.experimental.pallas import tpu as pltpu
from jax.experimental.pallas import tpu_sc as plsc

_HIDDEN = 64
_LANES = 16
_TH = 8
_TB = 128

_NC = 2
_NS = 16
_NW = _NC * _NS
_BLK = 128

_GR = 4
_WR = 2


def _body(nbatch, seq, table_hbm, labels_hbm, out_hbm,
          idx_v, r0, r1, r2, r3, ob0, ob1,
          gs0, gs1, gs2, gs3, ws0, ws1):
    wid = lax.axis_index("s") * _NC + lax.axis_index("c")

    rows = (r0, r1, r2, r3)
    obuf = (ob0, ob1)
    gsem = (gs0, gs1, gs2, gs3)
    wsem = (ws0, ws1)

    pltpu.sync_copy(labels_hbm.at[:, pl.ds(wid * _BLK, _BLK)], idx_v)

    iota = lax.broadcasted_iota(jnp.int32, (_LANES,), 0)
    ngrp = _BLK // _LANES
    rowidx = [iota + m * _LANES for m in range(ngrp)]

    def out_slice(s):
        return out_hbm.at[s, :, wid]

    def gather(s, r):
        return pltpu.async_copy(
            table_hbm.at[idx_v.at[s]], rows[r], gsem[r])

    def transpose_block(rv, ob):
        @plsc.parallel_loop(0, _HIDDEN, unroll=8)
        def h_body(h):
            th = h // _TH
            hh = h - th * _TH
            col = lax.broadcast(h, (_LANES,))
            for m in range(ngrp):
                v = plsc.load_gather(rv, [rowidx[m], col])
                ob[th, hh, pl.ds(m * _LANES, _LANES)] = v

    gather(0, 0)
    gather(1, 1)

    def step(j, carry):
        for b in range(_GR):
            s = _GR * j + b
            wb = b % _WR
            @pl.when(s + 2 < seq)
            def _fire():
                gather(s + 2, (b + 2) % _GR)
            pltpu.make_async_copy(
                table_hbm.at[idx_v.at[s]], rows[b], gsem[b]).wait()
            @pl.when(s >= _WR)
            def _drain():
                pltpu.make_async_copy(
                    obuf[wb], out_slice(s - _WR), wsem[wb]).wait()
            transpose_block(rows[b], obuf[wb])
            pltpu.async_copy(obuf[wb], out_slice(s), wsem[wb])
        return carry

    lax.fori_loop(0, seq // _GR, step, 0)

    pltpu.make_async_copy(ob0, out_slice(seq - 2), ws0).wait()
    pltpu.make_async_copy(ob1, out_slice(seq - 1), ws1).wait()


def kernel(pos_embedding_matrix, position_labels):
    b, s = position_labels.shape
    assert b == _NW * _BLK and s % _GR == 0
    ntb = b // _TB

    labels_t = position_labels.astype(jnp.int32).T

    mesh = plsc.VectorSubcoreMesh(core_axis_name="c", subcore_axis_name="s")
    run = pl.kernel(
        functools.partial(_body, b, s),
        mesh=mesh,
        compiler_params=pltpu.CompilerParams(
            use_tc_tiling_on_sc=False, needs_layout_passes=False),
        out_type=jax.ShapeDtypeStruct(
            (s, _HIDDEN // _TH, ntb, _TH, _TB), jnp.float32),
        scratch_types=(
            [pltpu.VMEM((s, _BLK), jnp.int32)]
            + [pltpu.VMEM((_BLK, _HIDDEN), jnp.float32)] * _GR
            + [pltpu.VMEM((_HIDDEN // _TH, _TH, _TB), jnp.float32)] * _WR
            + [pltpu.SemaphoreType.DMA] * (_GR + _WR)
        ),
    )
    raw = run(pos_embedding_matrix, labels_t)
    return jnp.transpose(raw, (2, 4, 0, 1, 3)).reshape(b, s, _HIDDEN)


# speedup vs baseline: 2.7301x; 2.7301x over previous
import functools

import jax
import jax.numpy as jnp
from jax import lax
from jax.experimental import pallas as pl
from jax.experimental.pallas import tpu as pltpu
from jax.experimental.pallas import tpu_sc as plsc

_HIDDEN = 64
_LANES = 16
_TH = 8
_TB = 128

_NC = 2
_NS = 16
_NW = _NC * _NS
_BLK = 128

_GR = 4
_WR = 2


def _body(nbatch, seq, table_hbm, labels_hbm, out_hbm,
          idx_v, r0, r1, r2, r3, ob0, ob1,
          gs0, gs1, gs2, gs3, ws0, ws1):
    wid = lax.axis_index("s") * _NC + lax.axis_index("c")

    rows = (r0, r1, r2, r3)
    obuf = (ob0, ob1)
    gsem = (gs0, gs1, gs2, gs3)
    wsem = (ws0, ws1)

    pltpu.sync_copy(labels_hbm.at[:, pl.ds(wid * _BLK, _BLK)], idx_v)

    iota = lax.broadcasted_iota(jnp.int32, (_LANES,), 0)
    ngrp = _BLK // _LANES
    rowidx = [iota + m * _LANES for m in range(ngrp)]

    def out_slice(s):
        return out_hbm.at[s, :, wid]

    def gather(s, r):
        return pltpu.async_copy(
            table_hbm.at[idx_v.at[s]], rows[r], gsem[r])

    def transpose_block(rv, ob):
        @plsc.parallel_loop(0, _LANES, unroll=2)
        def d_body(d):
            rot = (iota + d) & (_LANES - 1)
            for hb in range(_HIDDEN // _LANES):
                hvec = rot + hb * _LANES
                th = hvec >> 3
                hh = hvec & (_TH - 1)
                for rb in range(ngrp):
                    v = plsc.load_gather(rv, [rowidx[rb], hvec])
                    plsc.store_scatter(ob, [th, hh, rowidx[rb]], v)

    gather(0, 0)
    gather(1, 1)

    def step(j, carry):
        for b in range(_GR):
            s = _GR * j + b
            wb = b % _WR
            @pl.when(s + 2 < seq)
            def _fire():
                gather(s + 2, (b + 2) % _GR)
            pltpu.make_async_copy(
                table_hbm.at[idx_v.at[s]], rows[b], gsem[b]).wait()
            @pl.when(s >= _WR)
            def _drain():
                pltpu.make_async_copy(
                    obuf[wb], out_slice(s - _WR), wsem[wb]).wait()
            transpose_block(rows[b], obuf[wb])
            pltpu.async_copy(obuf[wb], out_slice(s), wsem[wb])
        return carry

    lax.fori_loop(0, seq // _GR, step, 0)

    pltpu.make_async_copy(ob0, out_slice(seq - 2), ws0).wait()
    pltpu.make_async_copy(ob1, out_slice(seq - 1), ws1).wait()


def kernel(pos_embedding_matrix, position_labels):
    b, s = position_labels.shape
    assert b == _NW * _BLK and s % _GR == 0
    ntb = b // _TB

    labels_t = position_labels.astype(jnp.int32).T

    mesh = plsc.VectorSubcoreMesh(core_axis_name="c", subcore_axis_name="s")
    run = pl.kernel(
        functools.partial(_body, b, s),
        mesh=mesh,
        compiler_params=pltpu.CompilerParams(
            use_tc_tiling_on_sc=False, needs_layout_passes=False),
        out_type=jax.ShapeDtypeStruct(
            (s, _HIDDEN // _TH, ntb, _TH, _TB), jnp.float32),
        scratch_types=(
            [pltpu.VMEM((s, _BLK), jnp.int32)]
            + [pltpu.VMEM((_BLK, _HIDDEN), jnp.float32)] * _GR
            + [pltpu.VMEM((_HIDDEN // _TH, _TH, _TB), jnp.float32)] * _WR
            + [pltpu.SemaphoreType.DMA] * (_GR + _WR)
        ),
    )
    raw = run(pos_embedding_matrix, labels_t)
    return jnp.transpose(raw, (2, 4, 0, 1, 3)).reshape(b, s, _HIDDEN)
